# baseline (speedup 1.0000x reference)
import jax
import jax.numpy as jnp
from jax import lax
from jax.experimental import pallas as pl
from jax.experimental.pallas import tpu as pltpu
from jax.experimental.pallas import tpu_sc as plsc

B, L, D = 4, 8192, 1024
NC, NS = 2, 16
NW = NC * NS
P = L // NW
C = 8
STEPS = P // C
NBUF = 3
LANES = 16
DCH = D // LANES


def _body(x_hbm, W_hbm, pe_hbm, out_hbm, idx_v, pe_v, rows_v, lsem, ssem):
    wid = lax.axis_index("s") * NC + lax.axis_index("c")
    base = wid * P

    for b in range(B):
        pltpu.sync_copy(x_hbm.at[b, pl.ds(base, P)], idx_v.at[b])

    def issue_loads(s, q):
        l0 = base + s * C
        pltpu.async_copy(pe_hbm.at[pl.ds(l0, C)], pe_v.at[q], lsem.at[q])
        for b in range(B):
            pltpu.async_copy(
                W_hbm.at[idx_v.at[b, pl.ds(s * C, C)]],
                rows_v.at[q, pl.ds(b * C, C)],
                lsem.at[q],
            )

    def wait_loads(q):
        pltpu.make_async_copy(
            pe_hbm.at[pl.ds(base, C)], pe_v.at[q], lsem.at[q]
        ).wait()
        pltpu.make_async_copy(
            W_hbm.at[idx_v.at[0, pl.ds(0, B * C)]], rows_v.at[q], lsem.at[q]
        ).wait()

    def compute(q):
        def add_row(j, c2):
            for d in range(DCH):
                pv = pe_v[q, j, pl.ds(d * LANES, LANES)]
                for b in range(B):
                    r = b * C + j
                    rows_v[q, r, pl.ds(d * LANES, LANES)] = (
                        rows_v[q, r, pl.ds(d * LANES, LANES)] + pv
                    )
            return c2

        lax.fori_loop(0, C, add_row, 0)

    def issue_stores(s, q):
        l0 = base + s * C
        for b in range(B):
            pltpu.async_copy(
                rows_v.at[q, pl.ds(b * C, C)],
                out_hbm.at[b, pl.ds(l0, C)],
                ssem.at[q],
            )

    def wait_stores(q):
        pltpu.make_async_copy(
            rows_v.at[q], out_hbm.at[0, pl.ds(base, B * C)], ssem.at[q]
        ).wait()

    issue_loads(0, 0)
    issue_loads(1, 1)

    def step(s, carry):
        q = lax.rem(s, NBUF)
        qn = lax.rem(s + 2, NBUF)

        @pl.when(s + 2 < STEPS)
        def _():
            @pl.when(s >= 1)
            def _():
                wait_stores(qn)

            issue_loads(s + 2, qn)

        wait_loads(q)
        compute(q)
        issue_stores(s, q)
        return carry

    lax.fori_loop(0, STEPS, step, 0)

    wait_stores((STEPS - 3) % NBUF)
    wait_stores((STEPS - 2) % NBUF)
    wait_stores((STEPS - 1) % NBUF)


_emb = pl.kernel(
    _body,
    out_type=jax.ShapeDtypeStruct((B, L, D), jnp.float32),
    mesh=plsc.VectorSubcoreMesh(core_axis_name="c", subcore_axis_name="s"),
    scratch_types=[
        pltpu.VMEM((B, P), jnp.int32),
        pltpu.VMEM((NBUF, C, D), jnp.float32),
        pltpu.VMEM((NBUF, B * C, D), jnp.float32),
        pltpu.SemaphoreType.DMA((NBUF,)),
        pltpu.SemaphoreType.DMA((NBUF,)),
    ],
)


def kernel(x, W, pe):
    return _emb(x, W, pe.reshape(L, D))


# speedup vs baseline: 1.2628x; 1.2628x over previous
import jax
import jax.numpy as jnp
from jax import lax
from jax.experimental import pallas as pl
from jax.experimental.pallas import tpu as pltpu
from jax.experimental.pallas import tpu_sc as plsc

B, L, D = 4, 8192, 1024
NC, NS = 2, 16
NW = NC * NS
P = L // NW
C = 8
STEPS = P // C
NBUF = 3
LANES = 16
DCH = D // LANES


def _body(x_hbm, W_hbm, pe_hbm, out_hbm, idx_v, pe_v, rows_v, lsem, ssem):
    wid = lax.axis_index("s") * NC + lax.axis_index("c")
    base = wid * P

    for b in range(B):
        pltpu.sync_copy(x_hbm.at[b, pl.ds(base, P)], idx_v.at[b])

    def issue_loads(s, q):
        l0 = base + s * C
        pltpu.async_copy(pe_hbm.at[pl.ds(l0, C)], pe_v.at[q], lsem.at[q])
        for b in range(B):
            pltpu.async_copy(
                W_hbm.at[idx_v.at[b, pl.ds(s * C, C)]],
                rows_v.at[q, pl.ds(b * C, C)],
                lsem.at[q],
            )

    def wait_loads(q):
        pltpu.make_async_copy(
            pe_hbm.at[pl.ds(base, C)], pe_v.at[q], lsem.at[q]
        ).wait()
        pltpu.make_async_copy(
            W_hbm.at[idx_v.at[0, pl.ds(0, B * C)]], rows_v.at[q], lsem.at[q]
        ).wait()

    def compute(q):
        def add_row(j, c2):
            for d in range(DCH):
                pv = pe_v[q, j, pl.ds(d * LANES, LANES)]
                for b in range(B):
                    r = b * C + j
                    rows_v[q, r, pl.ds(d * LANES, LANES)] = (
                        rows_v[q, r, pl.ds(d * LANES, LANES)] + pv
                    )
            return c2

        lax.fori_loop(0, C, add_row, 0)

    def issue_stores(s, q):
        l0 = base + s * C
        for b in range(B):
            pltpu.async_copy(
                rows_v.at[q, pl.ds(b * C, C)],
                out_hbm.at[b, pl.ds(l0, C)],
                ssem.at[q],
            )

    def wait_stores(q):
        pltpu.make_async_copy(
            rows_v.at[q], out_hbm.at[0, pl.ds(base, B * C)], ssem.at[q]
        ).wait()

    issue_loads(0, 0)
    issue_loads(1, 1)

    def step(s, carry):
        q = lax.rem(s, NBUF)
        qn = lax.rem(s + 2, NBUF)

        wait_loads(q)
        compute(q)
        issue_stores(s, q)

        @pl.when(s + 2 < STEPS)
        def _():
            @pl.when(s >= 1)
            def _():
                wait_stores(qn)

            issue_loads(s + 2, qn)

        return carry

    lax.fori_loop(0, STEPS, step, 0)

    wait_stores((STEPS - 3) % NBUF)
    wait_stores((STEPS - 2) % NBUF)
    wait_stores((STEPS - 1) % NBUF)


_emb = pl.kernel(
    _body,
    out_type=jax.ShapeDtypeStruct((B, L, D), jnp.float32),
    mesh=plsc.VectorSubcoreMesh(core_axis_name="c", subcore_axis_name="s"),
    scratch_types=[
        pltpu.VMEM((B, P), jnp.int32),
        pltpu.VMEM((NBUF, C, D), jnp.float32),
        pltpu.VMEM((NBUF, B * C, D), jnp.float32),
        pltpu.SemaphoreType.DMA((NBUF,)),
        pltpu.SemaphoreType.DMA((NBUF,)),
    ],
)


def kernel(x, W, pe):
    return _emb(x, W, pe.reshape(L, D))
